# Initial kernel scaffold; baseline (speedup 1.0000x reference)
import jax
import jax.numpy as jnp
from jax.experimental import pallas as pl


def kernel(residual, W_router, b_router, W_in, b_in, W_out, b_out):
    raise NotImplementedError("write your pallas kernel here")


# speedup vs baseline: 2.6487x; 2.6487x over previous
import functools

import jax
import jax.numpy as jnp
from jax.experimental import pallas as pl
from jax.experimental.pallas import tpu as pltpu

E = 8
TOP_K = 2
D = 768
F = 3072
T = 2048
FC = 768
NF = F // FC


def _moe_dense_body(x_ref, wr_ref, br_ref, win_ref, bin_ref, wout_ref, bout_ref,
                    out_ref, comb_ref, xbf_ref):
    e = pl.program_id(0)
    fc = pl.program_id(1)
    first = (e == 0) & (fc == 0)

    @pl.when(first)
    def _router():
        x = x_ref[...]
        xbf_ref[...] = x.astype(jnp.bfloat16)
        logits = jnp.dot(x, wr_ref[...], preferred_element_type=jnp.float32)
        logits = logits + br_ref[...]
        m = jnp.max(logits, axis=1, keepdims=True)
        p = jnp.exp(logits - m)
        p = p / jnp.sum(p, axis=1, keepdims=True)
        lane = jax.lax.broadcasted_iota(jnp.int32, p.shape, 1)
        i1 = jnp.argmax(p, axis=1, keepdims=True)
        t1 = jnp.max(p, axis=1, keepdims=True)
        p2 = jnp.where(lane == i1, -1.0, p)
        i2 = jnp.argmax(p2, axis=1, keepdims=True)
        t2 = jnp.max(p2, axis=1, keepdims=True)
        s = t1 + t2
        comb_ref[...] = jnp.where(lane == i1, t1 / s,
                                  jnp.where(lane == i2, t2 / s, 0.0))

    lane = jax.lax.broadcasted_iota(jnp.int32, (T, E), 1)
    wcol = jnp.sum(comb_ref[...] * (lane == e).astype(jnp.float32), axis=1,
                   keepdims=True)

    xb = xbf_ref[...]
    h = jnp.dot(xb, win_ref[0].astype(jnp.bfloat16),
                preferred_element_type=jnp.float32)
    h = h + bin_ref[0]
    h = jax.nn.gelu(h, approximate=True)
    y = jnp.dot(h.astype(jnp.bfloat16), wout_ref[0].astype(jnp.bfloat16),
                preferred_element_type=jnp.float32)
    y = y + jnp.where(fc == 0, 1.0, 0.0) * bout_ref[0]
    contrib = wcol * y

    @pl.when(first)
    def _init():
        out_ref[...] = contrib

    @pl.when(~first)
    def _acc():
        out_ref[...] += contrib


def _moe_dense(x, W_router, b_router, W_in, b_in, W_out, b_out):
    return pl.pallas_call(
        _moe_dense_body,
        grid=(E, NF),
        in_specs=[
            pl.BlockSpec((T, D), lambda e, f: (0, 0)),
            pl.BlockSpec((D, E), lambda e, f: (0, 0)),
            pl.BlockSpec((1, E), lambda e, f: (0, 0)),
            pl.BlockSpec((1, D, FC), lambda e, f: (e, 0, f)),
            pl.BlockSpec((1, 1, FC), lambda e, f: (e, 0, f)),
            pl.BlockSpec((1, FC, D), lambda e, f: (e, f, 0)),
            pl.BlockSpec((1, 1, D), lambda e, f: (e, 0, 0)),
        ],
        out_specs=pl.BlockSpec((T, D), lambda e, f: (0, 0)),
        out_shape=jax.ShapeDtypeStruct((T, D), jnp.float32),
        scratch_shapes=[
            pltpu.VMEM((T, E), jnp.float32),
            pltpu.VMEM((T, D), jnp.bfloat16),
        ],
        compiler_params=pltpu.CompilerParams(
            dimension_semantics=("arbitrary", "arbitrary"),
        ),
    )(x, W_router, b_router, W_in, b_in, W_out, b_out)


def kernel(residual, W_router, b_router, W_in, b_in, W_out, b_out):
    Bt, St, Dm = residual.shape
    x = residual.reshape(-1, Dm)
    out = _moe_dense(x, W_router, b_router.reshape(1, E), W_in,
                     b_in.reshape(E, 1, F), W_out, b_out.reshape(E, 1, Dm))
    return out.reshape(Bt, St, Dm)
